# Initial kernel scaffold; baseline (speedup 1.0000x reference)
import functools

import jax
import jax.numpy as jnp
from jax.experimental import pallas as pl
from jax.experimental.pallas import tpu as pltpu

N_TOK = 2048
D_MODEL = 1024
D_FF = 2048
N_EXP = 8
HID = 128

FF_BLK = 1024
N_FF_BLK = D_FF // FF_BLK


def _router_kernel(x_ref, rW1_ref, rb1_ref, rW2_ref, rb2_ref, cw_ref):
    x = x_ref[...]
    h = jnp.dot(x, rW1_ref[...], preferred_element_type=jnp.float32)
    h = jnp.maximum(h + rb1_ref[...], 0.0)
    logits = jnp.dot(h, rW2_ref[...], preferred_element_type=jnp.float32)
    logits = logits + rb2_ref[...]
    col = jax.lax.broadcasted_iota(jnp.int32, logits.shape, 1)
    l0 = jnp.max(logits, axis=1, keepdims=True)
    e0 = jnp.min(jnp.where(logits == l0, col, N_EXP), axis=1, keepdims=True)
    masked = jnp.where(col == e0, -jnp.inf, logits)
    l1 = jnp.max(masked, axis=1, keepdims=True)
    e1 = jnp.min(jnp.where(masked == l1, col, N_EXP), axis=1, keepdims=True)
    w0 = 1.0 / (1.0 + jnp.exp(l1 - l0))
    w1 = 1.0 - w0
    cw_ref[...] = jnp.where(col == e0, w0, 0.0) + jnp.where(col == e1, w1, 0.0)


def _ffn_kernel(x_ref, eW1_ref, eb1_ref, eW2_ref, eb2_ref, cw_ref,
                out_ref, xb_ref):
    e = pl.program_id(0)
    f = pl.program_id(1)

    @pl.when(jnp.logical_and(e == 0, f == 0))
    def _init():
        xb_ref[...] = x_ref[...].astype(jnp.bfloat16)
        out_ref[...] = jnp.zeros_like(out_ref)

    w1 = eW1_ref[0].astype(jnp.bfloat16)
    h = jnp.dot(xb_ref[...], w1, preferred_element_type=jnp.float32)
    h = jnp.maximum(h + eb1_ref[...], 0.0).astype(jnp.bfloat16)
    w2 = eW2_ref[0].astype(jnp.bfloat16)
    y = jnp.dot(h, w2, preferred_element_type=jnp.float32)
    y = y + jnp.where(f == N_FF_BLK - 1, 1.0, 0.0) * eb2_ref[...]
    col = jax.lax.broadcasted_iota(jnp.int32, (N_TOK, N_EXP), 1)
    wcol = jnp.sum(jnp.where(col == e, cw_ref[...], 0.0), axis=1, keepdims=True)
    out_ref[...] += y * wcol


def kernel(x, rW1, rb1, rW2, rb2, eW1, eb1, eW2, eb2):
    cw = pl.pallas_call(
        _router_kernel,
        out_shape=jax.ShapeDtypeStruct((N_TOK, N_EXP), jnp.float32),
    )(x, rW1, rb1.reshape(1, HID), rW2, rb2.reshape(1, N_EXP))

    out = pl.pallas_call(
        _ffn_kernel,
        grid=(N_EXP, N_FF_BLK),
        in_specs=[
            pl.BlockSpec((N_TOK, D_MODEL), lambda e, f: (0, 0)),
            pl.BlockSpec((1, D_MODEL, FF_BLK), lambda e, f: (e, 0, f)),
            pl.BlockSpec((1, FF_BLK), lambda e, f: (e, f)),
            pl.BlockSpec((1, FF_BLK, D_MODEL), lambda e, f: (e, f, 0)),
            pl.BlockSpec((1, D_MODEL), lambda e, f: (e, 0)),
            pl.BlockSpec((N_TOK, N_EXP), lambda e, f: (0, 0)),
        ],
        out_specs=pl.BlockSpec((N_TOK, D_MODEL), lambda e, f: (0, 0)),
        out_shape=jax.ShapeDtypeStruct((N_TOK, D_MODEL), jnp.float32),
        scratch_shapes=[pltpu.VMEM((N_TOK, D_MODEL), jnp.bfloat16)],
    )(x, eW1, eb1, eW2, eb2, cw)
    return out


# speedup vs baseline: 1.5958x; 1.5958x over previous
import functools

import jax
import jax.numpy as jnp
from jax.experimental import pallas as pl
from jax.experimental.pallas import tpu as pltpu

N_TOK = 2048
D_MODEL = 1024
D_FF = 2048
N_EXP = 8
HID = 128

FF_BLK = 1024
N_FF_BLK = D_FF // FF_BLK


def _router_kernel(x_ref, rW1_ref, rb1_ref, rW2_ref, rb2_ref, cw_ref):
    x = x_ref[...]
    h = jnp.dot(x, rW1_ref[...], preferred_element_type=jnp.float32)
    h = jnp.maximum(h + rb1_ref[...], 0.0)
    logits = jnp.dot(h, rW2_ref[...], preferred_element_type=jnp.float32)
    logits = logits + rb2_ref[...]
    col = jax.lax.broadcasted_iota(jnp.int32, logits.shape, 1)
    l0 = jnp.max(logits, axis=1, keepdims=True)
    e0 = jnp.min(jnp.where(logits == l0, col, N_EXP), axis=1, keepdims=True)
    masked = jnp.where(col == e0, -jnp.inf, logits)
    l1 = jnp.max(masked, axis=1, keepdims=True)
    e1 = jnp.min(jnp.where(masked == l1, col, N_EXP), axis=1, keepdims=True)
    w0 = 1.0 / (1.0 + jnp.exp(l1 - l0))
    w1 = 1.0 - w0
    cw_ref[...] = jnp.where(col == e0, w0, 0.0) + jnp.where(col == e1, w1, 0.0)


def _ffn_kernel(x_ref, eW1_ref, eb1_ref, eW2_ref, eb2_ref, cw_ref,
                out_ref, xb_ref):
    e = pl.program_id(0)
    f = pl.program_id(1)

    @pl.when(jnp.logical_and(e == 0, f == 0))
    def _init():
        xb_ref[...] = x_ref[...].astype(jnp.bfloat16)
        out_ref[...] = jnp.zeros_like(out_ref)

    w1 = eW1_ref[0].astype(jnp.bfloat16)
    h = jnp.dot(xb_ref[...], w1, preferred_element_type=jnp.float32)
    h = jnp.maximum(h + eb1_ref[0], 0.0).astype(jnp.bfloat16)
    w2 = eW2_ref[0].astype(jnp.bfloat16)
    y = jnp.dot(h, w2, preferred_element_type=jnp.float32)
    y = y + jnp.where(f == N_FF_BLK - 1, 1.0, 0.0) * eb2_ref[0]
    col = jax.lax.broadcasted_iota(jnp.int32, (N_TOK, N_EXP), 1)
    wcol = jnp.sum(jnp.where(col == e, cw_ref[...], 0.0), axis=1, keepdims=True)
    out_ref[...] += y * wcol


def kernel(x, rW1, rb1, rW2, rb2, eW1, eb1, eW2, eb2):
    cw = pl.pallas_call(
        _router_kernel,
        out_shape=jax.ShapeDtypeStruct((N_TOK, N_EXP), jnp.float32),
    )(x, rW1, rb1.reshape(1, HID), rW2, rb2.reshape(1, N_EXP))

    out = pl.pallas_call(
        _ffn_kernel,
        grid=(N_EXP, N_FF_BLK),
        in_specs=[
            pl.BlockSpec((N_TOK, D_MODEL), lambda e, f: (0, 0)),
            pl.BlockSpec((1, D_MODEL, FF_BLK), lambda e, f: (e, 0, f)),
            pl.BlockSpec((1, 1, FF_BLK), lambda e, f: (e, 0, f)),
            pl.BlockSpec((1, FF_BLK, D_MODEL), lambda e, f: (e, f, 0)),
            pl.BlockSpec((1, 1, D_MODEL), lambda e, f: (e, 0, 0)),
            pl.BlockSpec((N_TOK, N_EXP), lambda e, f: (0, 0)),
        ],
        out_specs=pl.BlockSpec((N_TOK, D_MODEL), lambda e, f: (0, 0)),
        out_shape=jax.ShapeDtypeStruct((N_TOK, D_MODEL), jnp.float32),
        scratch_shapes=[pltpu.VMEM((N_TOK, D_MODEL), jnp.bfloat16)],
    )(x, eW1, eb1.reshape(N_EXP, 1, D_FF), eW2, eb2.reshape(N_EXP, 1, D_MODEL), cw)
    return out
